# baseline (speedup 1.0000x reference)
import functools

import jax
import jax.numpy as jnp
from jax import lax
from jax.experimental import pallas as pl
from jax.experimental.pallas import tpu as pltpu
from jax.experimental.pallas import tpu_sc as plsc

VOCAB = 1000000
B = 16384
D = 16
NC = 2
NS = 16
NW = NC * NS
BPW = B // NW
L = 16
SR = 128 // D
C = 128
NCHUNK = BPW // C

_mesh = plsc.VectorSubcoreMesh(core_axis_name="c", subcore_axis_name="s")

_dnums = lax.GatherDimensionNumbers(
    offset_dims=(), collapsed_slice_dims=(0,), start_index_map=(0,))


def _vgather(vec, pat):
    return lax.gather(vec, pat[:, None], _dnums, slice_sizes=(1,),
                      mode=lax.GatherScatterMode.PROMISE_IN_BOUNDS)


@functools.partial(
    pl.kernel,
    mesh=_mesh,
    out_type=[
        jax.ShapeDtypeStruct((B * D,), jnp.float32),
        jax.ShapeDtypeStruct((B * D,), jnp.float32),
        jax.ShapeDtypeStruct((B * D,), jnp.float32),
    ],
    scratch_types=[
        pltpu.VMEM((2 * BPW,), jnp.int32),
        pltpu.VMEM((BPW,), jnp.int32),
        pltpu.VMEM((BPW,), jnp.int32),
        pltpu.VMEM((BPW,), jnp.int32),
        pltpu.VMEM((BPW,), jnp.int32),
        pltpu.VMEM((C, 128), jnp.float32),
        pltpu.VMEM((C, 128), jnp.float32),
        pltpu.VMEM((C, 128), jnp.float32),
        pltpu.VMEM((C, 128), jnp.float32),
        pltpu.VMEM((BPW * D,), jnp.float32),
        pltpu.VMEM((BPW * D,), jnp.float32),
        pltpu.SemaphoreType.DMA,
        pltpu.SemaphoreType.DMA,
        pltpu.SemaphoreType.DMA,
        pltpu.SemaphoreType.DMA,
        pltpu.SemaphoreType.DMA,
    ],
    compiler_params=pltpu.CompilerParams(use_tc_tiling_on_sc=True),
)
def _emb_lookup(ids_hbm, table_hbm, out1, out2, out3,
                pairs_v, idx1_v, idx2_v, sup1_v, sup2_v,
                r1a, r1b, r2a, r2b, stage1_v, stage2_v,
                sa, sb, sc_, sd, semo):
    wid = lax.axis_index("s") * NC + lax.axis_index("c")
    base = pl.multiple_of(wid * BPW, BPW)
    pltpu.sync_copy(ids_hbm.at[pl.ds(2 * base, 2 * BPW)], pairs_v)

    lane = lax.iota(jnp.int32, L)
    ev_pat = (2 * lane) & (L - 1)
    od_pat = (2 * lane + 1) & (L - 1)
    lo_half = lane < (L // 2)
    for j in range(BPW // L):
        a = pairs_v[pl.ds(2 * j * L, L)]
        b = pairs_v[pl.ds(2 * j * L + L, L)]
        i1 = jnp.where(lo_half, _vgather(a, ev_pat), _vgather(b, ev_pat))
        i2 = jnp.where(lo_half, _vgather(a, od_pat), _vgather(b, od_pat))
        idx1_v[pl.ds(j * L, L)] = i1
        idx2_v[pl.ds(j * L, L)] = i2
        sup1_v[pl.ds(j * L, L)] = lax.shift_right_logical(i1, 3)
        sup2_v[pl.ds(j * L, L)] = lax.shift_right_logical(i2, 3)

    bufs = [(r1a, r2a, sa, sc_), (r1b, r2b, sb, sd)]

    def issue(c):
        b1, b2, s1, s2 = bufs[c % 2]
        g1 = pltpu.async_copy(
            table_hbm.at[sup1_v.at[pl.ds(c * C, C)]], b1, s1)
        g2 = pltpu.async_copy(
            table_hbm.at[sup2_v.at[pl.ds(c * C, C)]], b2, s2)
        return g1, g2

    def extract(rows_v, idx_v, stage_v, c):
        def body(t, _):
            iv = idx_v[pl.ds(c * C + t * L, L)]
            gv = (iv & (SR - 1)) * D
            for r in range(L):
                k_local = t * L + r
                val = rows_v[k_local, pl.ds(gv[r], D)]
                stage_v[pl.ds((c * C + k_local) * D, D)] = val
            return 0

        lax.fori_loop(0, C // L, body, 0)

    pend = issue(0)
    for c in range(NCHUNK):
        nxt_pend = issue(c + 1) if c + 1 < NCHUNK else None
        g1, g2 = pend
        b1, b2, _, _ = bufs[c % 2]
        g1.wait()
        extract(b1, idx1_v, stage1_v, c)
        g2.wait()
        extract(b2, idx2_v, stage2_v, c)
        pend = nxt_pend

    o1 = pltpu.async_copy(stage1_v, out1.at[pl.ds(base * D, BPW * D)], semo)
    o2 = pltpu.async_copy(stage2_v, out2.at[pl.ds(base * D, BPW * D)], semo)
    o3 = pltpu.async_copy(stage2_v.at[pl.ds(0, (BPW - 1) * D)],
                          out3.at[pl.ds((base + 1) * D, (BPW - 1) * D)], semo)
    nxt = lax.rem(base + BPW, B)
    o4 = pltpu.async_copy(stage2_v.at[pl.ds((BPW - 1) * D, D)],
                          out3.at[pl.ds(nxt * D, D)], semo)
    o1.wait()
    o2.wait()
    o3.wait()
    o4.wait()


def kernel(input_ids, node_embedding):
    ids_flat = input_ids.reshape(2 * B)
    table_sr = node_embedding.reshape(VOCAB // SR, SR * D)
    f1, f2, f3 = _emb_lookup(ids_flat, table_sr)
    return f1.reshape(B, D), f2.reshape(B, D), f3.reshape(B, D)


# speedup vs baseline: 1.0190x; 1.0190x over previous
import functools

import jax
import jax.numpy as jnp
from jax import lax
from jax.experimental import pallas as pl
from jax.experimental.pallas import tpu as pltpu
from jax.experimental.pallas import tpu_sc as plsc

B = 16384
D = 16
NC = 2
NS = 16
NW = NC * NS
BPW = B // NW
L = 16

_mesh = plsc.VectorSubcoreMesh(core_axis_name="c", subcore_axis_name="s")


@functools.partial(
    pl.kernel,
    mesh=_mesh,
    out_type=[
        jax.ShapeDtypeStruct((B, D), jnp.float32),
        jax.ShapeDtypeStruct((B, D), jnp.float32),
        jax.ShapeDtypeStruct((B, D), jnp.float32),
    ],
    scratch_types=[
        pltpu.VMEM((2 * BPW,), jnp.int32),
        pltpu.VMEM((BPW,), jnp.int32),
        pltpu.VMEM((BPW,), jnp.int32),
        pltpu.VMEM((BPW, D), jnp.float32),
        pltpu.VMEM((BPW, D), jnp.float32),
        pltpu.SemaphoreType.DMA,
        pltpu.SemaphoreType.DMA,
        pltpu.SemaphoreType.DMA,
    ],
    compiler_params=pltpu.CompilerParams(use_tc_tiling_on_sc=False),
)
def _emb_lookup(ids_hbm, table_hbm, out1, out2, out3,
                pairs_v, idx1_v, idx2_v, rows1_v, rows2_v,
                sem1, sem2, semo):
    wid = lax.axis_index("s") * NC + lax.axis_index("c")
    base = pl.multiple_of(wid * BPW, BPW)
    pltpu.sync_copy(ids_hbm.at[pl.ds(2 * base, 2 * BPW)], pairs_v)
    lane = lax.iota(jnp.int32, L)
    ev_pat = (2 * lane) & (L - 1)
    od_pat = (2 * lane + 1) & (L - 1)
    lo_half = lane < (L // 2)
    dnums = lax.GatherDimensionNumbers(
        offset_dims=(), collapsed_slice_dims=(0,), start_index_map=(0,))

    def _vgather(vec, pat):
        return lax.gather(vec, pat[:, None], dnums, slice_sizes=(1,),
                          mode=lax.GatherScatterMode.PROMISE_IN_BOUNDS)

    for j in range(BPW // L):
        a = pairs_v[pl.ds(2 * j * L, L)]
        b = pairs_v[pl.ds(2 * j * L + L, L)]
        idx1_v[pl.ds(j * L, L)] = jnp.where(
            lo_half, _vgather(a, ev_pat), _vgather(b, ev_pat))
        idx2_v[pl.ds(j * L, L)] = jnp.where(
            lo_half, _vgather(a, od_pat), _vgather(b, od_pat))

    g1 = pltpu.async_copy(table_hbm.at[idx1_v], rows1_v, sem1)
    g2 = pltpu.async_copy(table_hbm.at[idx2_v], rows2_v, sem2)
    g1.wait()
    o1 = pltpu.async_copy(rows1_v, out1.at[pl.ds(base, BPW)], semo)
    g2.wait()
    o2 = pltpu.async_copy(rows2_v, out2.at[pl.ds(base, BPW)], semo)
    o3 = pltpu.async_copy(rows2_v.at[pl.ds(0, BPW - 1)],
                          out3.at[pl.ds(base + 1, BPW - 1)], semo)
    nxt = lax.rem(base + BPW, B)
    o4 = pltpu.async_copy(rows2_v.at[pl.ds(BPW - 1, 1)],
                          out3.at[pl.ds(nxt, 1)], semo)
    o1.wait()
    o2.wait()
    o3.wait()
    o4.wait()


def kernel(input_ids, node_embedding):
    ids_flat = input_ids.reshape(2 * B)
    out1, out2, out3 = _emb_lookup(ids_flat, node_embedding)
    return out1, out2, out3
